# baseline (speedup 1.0000x reference)
import functools

import jax
import jax.numpy as jnp
from jax import lax
from jax.experimental import pallas as pl
from jax.experimental.pallas import tpu as pltpu
from jax.experimental.pallas import tpu_sc as plsc


@functools.lru_cache(maxsize=None)
def _build(V, D, B):
    info = plsc.get_sparse_core_info()
    NC, NS = info.num_cores, info.num_subcores
    NW = NC * NS
    assert B % NW == 0
    b_per_w = B // NW

    mesh = plsc.VectorSubcoreMesh(core_axis_name="c", subcore_axis_name="s")

    @functools.partial(
        pl.kernel,
        mesh=mesh,
        out_type=jax.ShapeDtypeStruct((B, D), jnp.float32),
        compiler_params=pltpu.CompilerParams(use_tc_tiling_on_sc=False),
        scratch_types=[
            pltpu.VMEM((b_per_w,), jnp.int32),
            pltpu.VMEM((b_per_w, D), jnp.float32),
            pltpu.SemaphoreType.DMA,
        ],
    )
    def gather_kernel(table_hbm, idx_hbm, out_hbm, idx_v, rows_v, sem):
        wid = lax.axis_index("s") * NC + lax.axis_index("c")
        base = wid * b_per_w
        pltpu.sync_copy(idx_hbm.at[pl.ds(base, b_per_w)], idx_v)
        pltpu.async_copy(table_hbm.at[idx_v], rows_v, sem).wait()
        pltpu.sync_copy(rows_v, out_hbm.at[pl.ds(base, b_per_w)])

    return gather_kernel


def kernel(z, idx):
    V, D = z.shape
    B = idx.shape[0]
    return _build(V, D, B)(z, idx.astype(jnp.int32))


# speedup vs baseline: 1.6474x; 1.6474x over previous
import functools

import jax
import jax.numpy as jnp
from jax import lax
from jax.experimental import pallas as pl
from jax.experimental.pallas import tpu as pltpu
from jax.experimental.pallas import tpu_sc as plsc


@functools.lru_cache(maxsize=None)
def _build(V, D, B):
    info = plsc.get_sparse_core_info()
    NC, NS = info.num_cores, info.num_subcores
    NW = NC * NS
    assert B % NW == 0 and D % 16 == 0
    b_per_w = B // NW

    mesh = plsc.VectorSubcoreMesh(core_axis_name="c", subcore_axis_name="s")

    @functools.partial(
        pl.kernel,
        mesh=mesh,
        out_type=jax.ShapeDtypeStruct((D, B), jnp.float32),
        compiler_params=pltpu.CompilerParams(needs_layout_passes=False),
        scratch_types=[
            pltpu.VMEM((b_per_w,), jnp.int32),
            pltpu.VMEM((b_per_w, D), jnp.float32),
            pltpu.VMEM((D, b_per_w), jnp.float32),
            pltpu.SemaphoreType.DMA,
        ],
    )
    def gather_kernel(table_hbm, idx_hbm, out_hbm, idx_v, rows_v,
                      rowsT_v, sem_g):
        wid = lax.axis_index("s") * NC + lax.axis_index("c")
        base = wid * b_per_w
        pltpu.sync_copy(idx_hbm.at[pl.ds(base, b_per_w)], idx_v)
        iota16 = lax.iota(jnp.int32, 16)

        def fire_group(g, _):
            vec = idx_v[pl.ds(g * 16, 16)]
            for lane in range(16):
                i = jnp.max(jnp.where(iota16 == lane, vec, 0))
                pltpu.make_async_copy(
                    table_hbm.at[pl.ds(i, 1), :],
                    rows_v.at[pl.ds(g * 16 + lane, 1), :],
                    sem_g,
                ).start()
            return ()

        lax.fori_loop(0, b_per_w // 16, fire_group, (), unroll=False)
        pltpu.make_async_copy(
            table_hbm.at[pl.ds(0, b_per_w), :], rows_v, sem_g
        ).wait()

        iota = lax.iota(jnp.int32, 16)

        def feature(j, _):
            jv = jnp.zeros((16,), jnp.int32) + j
            for t in range(b_per_w // 16):
                vals = plsc.load_gather(rows_v, [t * 16 + iota, jv])
                rowsT_v[j, pl.ds(t * 16, 16)] = vals
            return ()

        lax.fori_loop(0, D, feature, (), unroll=False)
        pltpu.sync_copy(rowsT_v, out_hbm.at[:, pl.ds(base, b_per_w)])

    return gather_kernel


def kernel(z, idx):
    V, D = z.shape
    B = idx.shape[0]
    outT = _build(V, D, B)(z, idx.astype(jnp.int32))
    return outT.T


# speedup vs baseline: 1.7167x; 1.0421x over previous
import functools

import jax
import jax.numpy as jnp
from jax import lax
from jax.experimental import pallas as pl
from jax.experimental.pallas import tpu as pltpu
from jax.experimental.pallas import tpu_sc as plsc


@functools.lru_cache(maxsize=None)
def _build(V, D, B):
    info = plsc.get_sparse_core_info()
    NC, NS = info.num_cores, info.num_subcores
    NW = NC * NS
    assert B % NW == 0 and D % 16 == 0
    b_per_w = B // NW

    mesh = plsc.VectorSubcoreMesh(core_axis_name="c", subcore_axis_name="s")

    @functools.partial(
        pl.kernel,
        mesh=mesh,
        out_type=jax.ShapeDtypeStruct((B, D), jnp.float32),
        scratch_types=[
            pltpu.VMEM((b_per_w,), jnp.int32),
            pltpu.VMEM((b_per_w, D), jnp.float32),
            pltpu.SemaphoreType.DMA,
        ],
    )
    def gather_kernel(table_hbm, idx_hbm, out_hbm, idx_v, rows_v, sem_g):
        wid = lax.axis_index("s") * NC + lax.axis_index("c")
        base = wid * b_per_w
        pltpu.sync_copy(idx_hbm.at[pl.ds(base, b_per_w)], idx_v)

        def fire_group(g, _):
            vec = idx_v[pl.ds(g * 16, 16)]
            for lane in range(16):
                i = vec[lane]
                pltpu.make_async_copy(
                    table_hbm.at[pl.ds(i, 1), :],
                    rows_v.at[pl.ds(g * 16 + lane, 1), :],
                    sem_g,
                ).start()
            return ()

        lax.fori_loop(0, b_per_w // 16, fire_group, (), unroll=False)
        pltpu.make_async_copy(
            table_hbm.at[pl.ds(0, b_per_w), :], rows_v, sem_g
        ).wait()
        pltpu.sync_copy(rows_v, out_hbm.at[pl.ds(base, b_per_w), :])

    return gather_kernel


def kernel(z, idx):
    V, D = z.shape
    B = idx.shape[0]
    return _build(V, D, B)(z, idx.astype(jnp.int32))
